# baseline (speedup 1.0000x reference)
import functools

import jax
import jax.numpy as jnp
from jax import lax
from jax.experimental import pallas as pl
from jax.experimental.pallas import tpu as pltpu
from jax.experimental.pallas import tpu_sc as plsc

N = 10000
NP = 10240
E = 160000
CH = 128
NCH1 = 80
NCH2 = 40
EP = 16 * NCH1 * CH
STRIPE = NP // 16

_MESH = plsc.VectorSubcoreMesh(
    core_axis_name="c", subcore_axis_name="s", num_cores=2, num_subcores=16)


@functools.partial(
    pl.kernel,
    out_type=jax.ShapeDtypeStruct((2, NP), jnp.float32),
    mesh=_MESH,
    scratch_types=[
        pltpu.VMEM((NCH2, CH), jnp.int32),
        pltpu.VMEM((CH,), jnp.float32),
        pltpu.VMEM_SHARED((NP,), jnp.float32),
    ],
)
def _deg(col_hbm, zeros1_hbm, out_hbm, col_v, ones_v, deg_sh):
    c = lax.axis_index("c")
    s = lax.axis_index("s")
    pltpu.sync_copy(col_hbm.at[c, s], col_v)

    @pl.loop(0, CH // 16)
    def _fill(i):
        ones_v[pl.ds(i * 16, 16)] = jnp.full((16,), 1.0, jnp.float32)

    pltpu.sync_copy(zeros1_hbm.at[pl.ds(s * STRIPE, STRIPE)],
                    deg_sh.at[pl.ds(s * STRIPE, STRIPE)])
    plsc.subcore_barrier()

    @pl.loop(0, NCH2)
    def _scat(g):
        pltpu.sync_copy(ones_v, deg_sh.at[col_v.at[g]], add=True)

    plsc.subcore_barrier()
    pltpu.sync_copy(deg_sh.at[pl.ds(s * STRIPE, STRIPE)],
                    out_hbm.at[c, pl.ds(s * STRIPE, STRIPE)])


def _make_spmm(n_chunks, fw, params=None):
    @functools.partial(
        pl.kernel,
        out_type=jax.ShapeDtypeStruct((2, NP, fw), jnp.float32),
        mesh=_MESH,
        scratch_types=[
            pltpu.VMEM((n_chunks, CH), jnp.int32),
            pltpu.VMEM((n_chunks, CH), jnp.int32),
            pltpu.VMEM((CH, fw), jnp.float32),
            pltpu.SemaphoreType.DMA,
            pltpu.VMEM_SHARED((NP, fw), jnp.float32),
        ],
        compiler_params=params,
    )
    def spmm(row_hbm, col_hbm, table_hbm, out_hbm,
             row_v, col_v, buf, gsem, acc_sh):
        c = lax.axis_index("c")
        s = lax.axis_index("s")
        pltpu.sync_copy(row_hbm.at[c, s], row_v)
        pltpu.sync_copy(col_hbm.at[c, s], col_v)
        pltpu.sync_copy(table_hbm.at[pl.ds(c * NP + s * STRIPE, STRIPE)],
                        acc_sh.at[pl.ds(s * STRIPE, STRIPE)])
        plsc.subcore_barrier()

        @pl.loop(0, n_chunks)
        def _chunk(g):
            pltpu.async_copy(table_hbm.at[row_v.at[g]], buf, gsem).wait()
            pltpu.sync_copy(buf, acc_sh.at[col_v.at[g]], add=True)

        plsc.subcore_barrier()
        pltpu.sync_copy(acc_sh.at[pl.ds(s * STRIPE, STRIPE)],
                        out_hbm.at[c, pl.ds(s * STRIPE, STRIPE)])

    return spmm


_spmm1 = _make_spmm(NCH1, 128)
_spmm2 = _make_spmm(NCH1, 64,
                    pltpu.CompilerParams(use_tc_tiling_on_sc=False))


def _mm1_body(x_ref, w_ref, dega_ref, degb_ref, ht0_ref, ht1_ref, dinv_ref):
    deg = dega_ref[...] + degb_ref[...] + 1.0
    d = jnp.broadcast_to(lax.rsqrt(deg), (128, 128))
    h = jnp.dot(x_ref[...], w_ref[...], preferred_element_type=jnp.float32)
    ht0_ref[...] = h[:, :128] * d
    ht1_ref[...] = h[:, 128:] * d
    dinv_ref[...] = d


def _mm1_prescale(xp, w1, dega, degb):
    return pl.pallas_call(
        _mm1_body,
        grid=(NP // 128,),
        in_specs=[pl.BlockSpec((128, 256), lambda i: (i, 0)),
                  pl.BlockSpec((256, 256), lambda i: (0, 0)),
                  pl.BlockSpec((128, 1), lambda i: (i, 0)),
                  pl.BlockSpec((128, 1), lambda i: (i, 0))],
        out_specs=[pl.BlockSpec((128, 128), lambda i: (i, 0))] * 3,
        out_shape=[jax.ShapeDtypeStruct((NP, 128), jnp.float32)] * 3,
    )(xp, w1, dega, degb)


def _mid_body(a0_ref, a1_ref, dinv_ref, b0_ref, b1_ref,
              w_ref, o0_ref, o1_ref):
    d = dinv_ref[...]
    r0 = jnp.maximum(a0_ref[...] * d + b0_ref[...], 0.0)
    r1 = jnp.maximum(a1_ref[...] * d + b1_ref[...], 0.0)
    w = w_ref[...]
    h2 = (jnp.dot(r0, w[:128, :], preferred_element_type=jnp.float32)
          + jnp.dot(r1, w[128:, :], preferred_element_type=jnp.float32))
    ht2 = h2 * d
    o0_ref[...] = ht2[:, :64]
    o1_ref[...] = ht2[:, 64:]


def _mid(a0, a1, dinv, b0r, b1r, w2):
    spec128 = pl.BlockSpec((128, 128), lambda i: (i, 0))
    spec64 = pl.BlockSpec((128, 64), lambda i: (i, 0))
    return pl.pallas_call(
        _mid_body,
        grid=(NP // 128,),
        in_specs=[spec128, spec128, spec128,
                  pl.BlockSpec((1, 128), lambda i: (0, 0)),
                  pl.BlockSpec((1, 128), lambda i: (0, 0)),
                  pl.BlockSpec((256, 128), lambda i: (0, 0))],
        out_specs=[spec64, spec64],
        out_shape=[jax.ShapeDtypeStruct((NP, 64), jnp.float32)] * 2,
    )(a0, a1, dinv, b0r, b1r, w2)


def _final_body(a0_ref, a1_ref, dinv_ref, b_ref, o_ref):
    d = dinv_ref[...]
    o_ref[...] = jnp.concatenate(
        [a0_ref[...] * d[:, :64], a1_ref[...] * d[:, 64:]],
        axis=1) + b_ref[...]


def _final(a0, a1, dinv, b2r):
    spec128 = pl.BlockSpec((128, 128), lambda i: (i, 0))
    spec64 = pl.BlockSpec((128, 64), lambda i: (i, 0))
    return pl.pallas_call(
        _final_body,
        grid=(NP // 128,),
        in_specs=[spec64, spec64, spec128,
                  pl.BlockSpec((1, 128), lambda i: (0, 0))],
        out_specs=spec128,
        out_shape=jax.ShapeDtypeStruct((NP, 128), jnp.float32),
    )(a0, a1, dinv, b2r)


def kernel(x, edge_index, W1, b1, W2, b2):
    ei = edge_index.astype(jnp.int32)
    pad = jnp.full((EP - E,), N, jnp.int32)
    rowp = jnp.concatenate([ei[0], pad])
    colp = jnp.concatenate([ei[1], pad])
    row_l1h = rowp.reshape(16, NCH1, CH)
    row_l1 = jnp.stack([row_l1h, row_l1h + NP])
    col_l1 = jnp.broadcast_to(colp.reshape(1, 16, NCH1, CH),
                              (2, 16, NCH1, CH))
    row_l2 = rowp.reshape(2, 16, NCH2, CH)
    col_l2 = colp.reshape(2, 16, NCH2, CH)

    zeros1 = jnp.zeros((NP,), jnp.float32)
    xp = jnp.pad(x, ((0, NP - N), (0, 0)))

    deg = _deg(col_l2, zeros1)
    ht0, ht1, dinv = _mm1_prescale(xp, W1, deg[0].reshape(NP, 1),
                                   deg[1].reshape(NP, 1))
    table1 = jnp.concatenate([ht0, ht1], axis=0)
    acc1 = _spmm1(row_l1, col_l1, table1)
    t2a, t2b = _mid(acc1[0], acc1[1], dinv,
                    b1[:128].reshape(1, 128), b1[128:].reshape(1, 128), W2)
    table2 = jnp.concatenate([t2a, t2b], axis=0)
    acc2 = _spmm2(row_l1, col_l1, table2)
    outp = _final(acc2[0], acc2[1], dinv, b2.reshape(1, 128))
    return outp[:N]


# speedup vs baseline: 1.0097x; 1.0097x over previous
import functools

import jax
import jax.numpy as jnp
from jax import lax
from jax.experimental import pallas as pl
from jax.experimental.pallas import tpu as pltpu
from jax.experimental.pallas import tpu_sc as plsc

N = 10000
NP = 10240
E = 160000
CH = 128
NCH1 = 80
NCH2 = 40
EP = 16 * NCH1 * CH
STRIPE = NP // 16

_MESH = plsc.VectorSubcoreMesh(
    core_axis_name="c", subcore_axis_name="s", num_cores=2, num_subcores=16)


@functools.partial(
    pl.kernel,
    out_type=jax.ShapeDtypeStruct((2, NP), jnp.float32),
    mesh=_MESH,
    scratch_types=[
        pltpu.VMEM((NCH2, CH), jnp.int32),
        pltpu.VMEM((CH,), jnp.float32),
        pltpu.VMEM_SHARED((NP,), jnp.float32),
    ],
)
def _deg(col_hbm, zeros1_hbm, out_hbm, col_v, ones_v, deg_sh):
    c = lax.axis_index("c")
    s = lax.axis_index("s")
    pltpu.sync_copy(col_hbm.at[c, s], col_v)

    @pl.loop(0, CH // 16)
    def _fill(i):
        ones_v[pl.ds(i * 16, 16)] = jnp.full((16,), 1.0, jnp.float32)

    pltpu.sync_copy(zeros1_hbm.at[pl.ds(s * STRIPE, STRIPE)],
                    deg_sh.at[pl.ds(s * STRIPE, STRIPE)])
    plsc.subcore_barrier()

    @pl.loop(0, NCH2)
    def _scat(g):
        pltpu.sync_copy(ones_v, deg_sh.at[col_v.at[g]], add=True)

    plsc.subcore_barrier()
    pltpu.sync_copy(deg_sh.at[pl.ds(s * STRIPE, STRIPE)],
                    out_hbm.at[c, pl.ds(s * STRIPE, STRIPE)])


def _make_spmm(n_chunks, fw, params=None):
    @functools.partial(
        pl.kernel,
        out_type=jax.ShapeDtypeStruct((2, NP, fw), jnp.float32),
        mesh=_MESH,
        scratch_types=[
            pltpu.VMEM((n_chunks, CH), jnp.int32),
            pltpu.VMEM((n_chunks, CH), jnp.int32),
            pltpu.VMEM((CH, fw), jnp.float32),
            pltpu.SemaphoreType.DMA,
            pltpu.VMEM_SHARED((NP, fw), jnp.float32),
        ],
        compiler_params=params,
    )
    def spmm(row_hbm, col_hbm, table_hbm, zeros2_hbm, out_hbm,
             row_v, col_v, buf, gsem, acc_sh):
        c = lax.axis_index("c")
        s = lax.axis_index("s")
        pltpu.sync_copy(row_hbm.at[c, s], row_v)
        pltpu.sync_copy(col_hbm.at[c, s], col_v)
        pltpu.sync_copy(zeros2_hbm, acc_sh.at[pl.ds(s * STRIPE, STRIPE)])
        plsc.subcore_barrier()

        @pl.loop(0, n_chunks)
        def _chunk(g):
            pltpu.async_copy(table_hbm.at[row_v.at[g]], buf, gsem).wait()
            pltpu.sync_copy(buf, acc_sh.at[col_v.at[g]], add=True)

        plsc.subcore_barrier()
        pltpu.sync_copy(acc_sh.at[pl.ds(s * STRIPE, STRIPE)],
                        out_hbm.at[c, pl.ds(s * STRIPE, STRIPE)])

    return spmm


_spmm1 = _make_spmm(NCH1, 128)
_spmm2 = _make_spmm(NCH1, 64,
                    pltpu.CompilerParams(use_tc_tiling_on_sc=False))


def _mm1_body(x_ref, w_ref, dega_ref, degb_ref, ht0_ref, ht1_ref, dinv_ref):
    deg = dega_ref[...] + degb_ref[...] + 1.0
    d = jnp.broadcast_to(lax.rsqrt(deg), (128, 128))
    h = jnp.dot(x_ref[...], w_ref[...], preferred_element_type=jnp.float32)
    ht0_ref[...] = h[:, :128] * d
    ht1_ref[...] = h[:, 128:] * d
    dinv_ref[...] = d


def _mm1_prescale(xp, w1, dega, degb):
    return pl.pallas_call(
        _mm1_body,
        grid=(NP // 128,),
        in_specs=[pl.BlockSpec((128, 256), lambda i: (i, 0)),
                  pl.BlockSpec((256, 256), lambda i: (0, 0)),
                  pl.BlockSpec((128, 1), lambda i: (i, 0)),
                  pl.BlockSpec((128, 1), lambda i: (i, 0))],
        out_specs=[pl.BlockSpec((128, 128), lambda i: (i, 0))] * 3,
        out_shape=[jax.ShapeDtypeStruct((NP, 128), jnp.float32)] * 3,
    )(xp, w1, dega, degb)


def _mid_body(a0_ref, a1_ref, t0_ref, t1_ref, dinv_ref, b0_ref, b1_ref,
              w_ref, o0_ref, o1_ref):
    d = dinv_ref[...]
    r0 = jnp.maximum((a0_ref[...] + t0_ref[...]) * d + b0_ref[...], 0.0)
    r1 = jnp.maximum((a1_ref[...] + t1_ref[...]) * d + b1_ref[...], 0.0)
    w = w_ref[...]
    h2 = (jnp.dot(r0, w[:128, :], preferred_element_type=jnp.float32)
          + jnp.dot(r1, w[128:, :], preferred_element_type=jnp.float32))
    ht2 = h2 * d
    o0_ref[...] = ht2[:, :64]
    o1_ref[...] = ht2[:, 64:]


def _mid(a0, a1, t0, t1, dinv, b0r, b1r, w2):
    spec128 = pl.BlockSpec((128, 128), lambda i: (i, 0))
    spec64 = pl.BlockSpec((128, 64), lambda i: (i, 0))
    return pl.pallas_call(
        _mid_body,
        grid=(NP // 128,),
        in_specs=[spec128, spec128, spec128, spec128, spec128,
                  pl.BlockSpec((1, 128), lambda i: (0, 0)),
                  pl.BlockSpec((1, 128), lambda i: (0, 0)),
                  pl.BlockSpec((256, 128), lambda i: (0, 0))],
        out_specs=[spec64, spec64],
        out_shape=[jax.ShapeDtypeStruct((NP, 64), jnp.float32)] * 2,
    )(a0, a1, t0, t1, dinv, b0r, b1r, w2)


def _final_body(a0_ref, a1_ref, t0_ref, t1_ref, dinv_ref, b_ref, o_ref):
    d = dinv_ref[...]
    o_ref[...] = jnp.concatenate(
        [(a0_ref[...] + t0_ref[...]) * d[:, :64],
         (a1_ref[...] + t1_ref[...]) * d[:, 64:]], axis=1) + b_ref[...]


def _final(a0, a1, t0, t1, dinv, b2r):
    spec128 = pl.BlockSpec((128, 128), lambda i: (i, 0))
    spec64 = pl.BlockSpec((128, 64), lambda i: (i, 0))
    return pl.pallas_call(
        _final_body,
        grid=(NP // 128,),
        in_specs=[spec64, spec64, spec64, spec64, spec128,
                  pl.BlockSpec((1, 128), lambda i: (0, 0))],
        out_specs=spec128,
        out_shape=jax.ShapeDtypeStruct((NP, 128), jnp.float32),
    )(a0, a1, t0, t1, dinv, b2r)


def kernel(x, edge_index, W1, b1, W2, b2):
    ei = edge_index.astype(jnp.int32)
    pad = jnp.full((EP - E,), N, jnp.int32)
    rowp = jnp.concatenate([ei[0], pad])
    colp = jnp.concatenate([ei[1], pad])
    row_l1h = rowp.reshape(16, NCH1, CH)
    row_l1 = jnp.stack([row_l1h, row_l1h + NP])
    col_l1 = jnp.broadcast_to(colp.reshape(1, 16, NCH1, CH),
                              (2, 16, NCH1, CH))
    row_l2 = rowp.reshape(2, 16, NCH2, CH)
    col_l2 = colp.reshape(2, 16, NCH2, CH)

    zeros1 = jnp.zeros((NP,), jnp.float32)
    zeros2 = jnp.zeros((STRIPE, 128), jnp.float32)
    zeros2h = jnp.zeros((STRIPE, 64), jnp.float32)
    xp = jnp.pad(x, ((0, NP - N), (0, 0)))

    deg = _deg(col_l2, zeros1)
    ht0, ht1, dinv = _mm1_prescale(xp, W1, deg[0].reshape(NP, 1),
                                   deg[1].reshape(NP, 1))
    table1 = jnp.concatenate([ht0, ht1], axis=0)
    acc1 = _spmm1(row_l1, col_l1, table1, zeros2)
    t2a, t2b = _mid(acc1[0], acc1[1], ht0, ht1, dinv,
                    b1[:128].reshape(1, 128), b1[128:].reshape(1, 128), W2)
    table2 = jnp.concatenate([t2a, t2b], axis=0)
    acc2 = _spmm2(row_l1, col_l1, table2, zeros2h)
    outp = _final(acc2[0], acc2[1], t2a, t2b, dinv, b2.reshape(1, 128))
    return outp[:N]


# speedup vs baseline: 1.0694x; 1.0591x over previous
import functools

import jax
import jax.numpy as jnp
from jax import lax
from jax.experimental import pallas as pl
from jax.experimental.pallas import tpu as pltpu
from jax.experimental.pallas import tpu_sc as plsc

N = 10000
NP = 10240
E = 160000
CH = 128
NCH1 = 80
NCH2 = 40
EP = 16 * NCH1 * CH
STRIPE = NP // 16

_MESH = plsc.VectorSubcoreMesh(
    core_axis_name="c", subcore_axis_name="s", num_cores=2, num_subcores=16)


@functools.partial(
    pl.kernel,
    out_type=jax.ShapeDtypeStruct((2, NP), jnp.float32),
    mesh=_MESH,
    scratch_types=[
        pltpu.VMEM((NCH2, CH), jnp.int32),
        pltpu.VMEM((CH,), jnp.float32),
        pltpu.VMEM_SHARED((NP,), jnp.float32),
    ],
)
def _deg(col_hbm, zeros1_hbm, out_hbm, col_v, ones_v, deg_sh):
    c = lax.axis_index("c")
    s = lax.axis_index("s")
    pltpu.sync_copy(col_hbm.at[c, s], col_v)

    @pl.loop(0, CH // 16)
    def _fill(i):
        ones_v[pl.ds(i * 16, 16)] = jnp.full((16,), 1.0, jnp.float32)

    pltpu.sync_copy(zeros1_hbm.at[pl.ds(s * STRIPE, STRIPE)],
                    deg_sh.at[pl.ds(s * STRIPE, STRIPE)])
    plsc.subcore_barrier()

    @pl.loop(0, NCH2)
    def _scat(g):
        pltpu.sync_copy(ones_v, deg_sh.at[col_v.at[g]], add=True)

    plsc.subcore_barrier()
    pltpu.sync_copy(deg_sh.at[pl.ds(s * STRIPE, STRIPE)],
                    out_hbm.at[c, pl.ds(s * STRIPE, STRIPE)])


def _make_spmm(n_chunks, fw, params=None):
    @functools.partial(
        pl.kernel,
        out_type=jax.ShapeDtypeStruct((2, NP, fw), jnp.float32),
        mesh=_MESH,
        scratch_types=[
            pltpu.VMEM((n_chunks, CH), jnp.int32),
            pltpu.VMEM((n_chunks, CH), jnp.int32),
            pltpu.VMEM((CH, fw), jnp.float32),
            pltpu.SemaphoreType.DMA,
            pltpu.VMEM_SHARED((NP, fw), jnp.float32),
        ],
        compiler_params=params,
    )
    def spmm(row_hbm, col_hbm, table_hbm, zeros2_hbm, out_hbm,
             row_v, col_v, buf, gsem, acc_sh):
        c = lax.axis_index("c")
        s = lax.axis_index("s")
        pltpu.sync_copy(row_hbm.at[c, s], row_v)
        pltpu.sync_copy(col_hbm.at[c, s], col_v)
        pltpu.sync_copy(zeros2_hbm, acc_sh.at[pl.ds(s * STRIPE, STRIPE)])
        plsc.subcore_barrier()

        @pl.loop(0, n_chunks)
        def _chunk(g):
            pltpu.async_copy(table_hbm.at[row_v.at[g]], buf, gsem).wait()
            pltpu.sync_copy(buf, acc_sh.at[col_v.at[g]], add=True)

        plsc.subcore_barrier()
        pltpu.sync_copy(acc_sh.at[pl.ds(s * STRIPE, STRIPE)],
                        out_hbm.at[c, pl.ds(s * STRIPE, STRIPE)])

    return spmm


_spmm1 = _make_spmm(NCH1, 128)


@functools.partial(
    pl.kernel,
    out_type=jax.ShapeDtypeStruct((2, NP, 64), jnp.float32),
    mesh=_MESH,
    scratch_types=[
        pltpu.VMEM((NCH1, CH), jnp.int32),
        pltpu.VMEM((NCH1, CH), jnp.int32),
        pltpu.VMEM((CH, 64), jnp.float32),
        pltpu.VMEM((64, 64), jnp.float32),
        pltpu.VMEM((STRIPE, 16), jnp.float32),
        pltpu.VMEM((64,), jnp.float32),
        pltpu.SemaphoreType.DMA,
        pltpu.VMEM_SHARED((NP, 64), jnp.float32),
    ],
    compiler_params=pltpu.CompilerParams(use_tc_tiling_on_sc=False),
)
def _spmm2_fused(row_hbm, col_hbm, table_hbm, dinv_hbm, b2_hbm, out_hbm,
                 row_v, col_v, buf, pa, dinv_v, b2v, gsem, acc_sh):
    c = lax.axis_index("c")
    s = lax.axis_index("s")
    pltpu.sync_copy(row_hbm.at[c, s], row_v)
    pltpu.sync_copy(col_hbm.at[c, s], col_v)
    pltpu.sync_copy(table_hbm.at[pl.ds(c * NP + s * STRIPE, STRIPE)],
                    acc_sh.at[pl.ds(s * STRIPE, STRIPE)])
    pltpu.sync_copy(dinv_hbm.at[pl.ds(s * STRIPE, STRIPE), pl.ds(0, 16)],
                    dinv_v)
    pltpu.sync_copy(b2_hbm.at[c], b2v)
    plsc.subcore_barrier()

    @pl.loop(0, NCH1)
    def _chunk(g):
        pltpu.async_copy(table_hbm.at[row_v.at[g]], buf, gsem).wait()
        pltpu.sync_copy(buf, acc_sh.at[col_v.at[g]], add=True)

    plsc.subcore_barrier()

    @pl.loop(0, STRIPE // 64)
    def _piece(k):
        base = s * STRIPE + k * 64
        pltpu.sync_copy(acc_sh.at[pl.ds(base, 64)], pa)

        @pl.loop(0, 64)
        def _row(r):
            dvec = dinv_v[k * 64 + r, pl.ds(0, 16)]
            for j in range(4):
                sl = pl.ds(j * 16, 16)
                pa[r, sl] = pa[r, sl] * dvec + b2v[sl]

        pltpu.sync_copy(pa, out_hbm.at[c, pl.ds(base, 64)])


def _mm1_body(x_ref, w_ref, dega_ref, degb_ref, ht0_ref, ht1_ref, dinv_ref):
    deg = dega_ref[...] + degb_ref[...] + 1.0
    d = jnp.broadcast_to(lax.rsqrt(deg), (128, 128))
    h = jnp.dot(x_ref[...], w_ref[...], preferred_element_type=jnp.float32)
    ht0_ref[...] = h[:, :128] * d
    ht1_ref[...] = h[:, 128:] * d
    dinv_ref[...] = d


def _mm1_prescale(xp, w1, dega, degb):
    return pl.pallas_call(
        _mm1_body,
        grid=(NP // 128,),
        in_specs=[pl.BlockSpec((128, 256), lambda i: (i, 0)),
                  pl.BlockSpec((256, 256), lambda i: (0, 0)),
                  pl.BlockSpec((128, 1), lambda i: (i, 0)),
                  pl.BlockSpec((128, 1), lambda i: (i, 0))],
        out_specs=[pl.BlockSpec((128, 128), lambda i: (i, 0))] * 3,
        out_shape=[jax.ShapeDtypeStruct((NP, 128), jnp.float32)] * 3,
    )(xp, w1, dega, degb)


def _mid_body(a0_ref, a1_ref, t0_ref, t1_ref, dinv_ref, b0_ref, b1_ref,
              w_ref, o0_ref, o1_ref):
    d = dinv_ref[...]
    r0 = jnp.maximum((a0_ref[...] + t0_ref[...]) * d + b0_ref[...], 0.0)
    r1 = jnp.maximum((a1_ref[...] + t1_ref[...]) * d + b1_ref[...], 0.0)
    w = w_ref[...]
    h2 = (jnp.dot(r0, w[:128, :], preferred_element_type=jnp.float32)
          + jnp.dot(r1, w[128:, :], preferred_element_type=jnp.float32))
    ht2 = h2 * d
    o0_ref[...] = ht2[:, :64]
    o1_ref[...] = ht2[:, 64:]


def _mid(a0, a1, t0, t1, dinv, b0r, b1r, w2):
    spec128 = pl.BlockSpec((128, 128), lambda i: (i, 0))
    spec64 = pl.BlockSpec((128, 64), lambda i: (i, 0))
    return pl.pallas_call(
        _mid_body,
        grid=(NP // 128,),
        in_specs=[spec128, spec128, spec128, spec128, spec128,
                  pl.BlockSpec((1, 128), lambda i: (0, 0)),
                  pl.BlockSpec((1, 128), lambda i: (0, 0)),
                  pl.BlockSpec((256, 128), lambda i: (0, 0))],
        out_specs=[spec64, spec64],
        out_shape=[jax.ShapeDtypeStruct((NP, 64), jnp.float32)] * 2,
    )(a0, a1, t0, t1, dinv, b0r, b1r, w2)


def _final_body(a0_ref, a1_ref, t0_ref, t1_ref, dinv_ref, b_ref, o_ref):
    d = dinv_ref[...]
    o_ref[...] = jnp.concatenate(
        [(a0_ref[...] + t0_ref[...]) * d[:, :64],
         (a1_ref[...] + t1_ref[...]) * d[:, 64:]], axis=1) + b_ref[...]


def _final(a0, a1, t0, t1, dinv, b2r):
    spec128 = pl.BlockSpec((128, 128), lambda i: (i, 0))
    spec64 = pl.BlockSpec((128, 64), lambda i: (i, 0))
    return pl.pallas_call(
        _final_body,
        grid=(NP // 128,),
        in_specs=[spec64, spec64, spec64, spec64, spec128,
                  pl.BlockSpec((1, 128), lambda i: (0, 0))],
        out_specs=spec128,
        out_shape=jax.ShapeDtypeStruct((NP, 128), jnp.float32),
    )(a0, a1, t0, t1, dinv, b2r)


def kernel(x, edge_index, W1, b1, W2, b2):
    ei = edge_index.astype(jnp.int32)
    pad = jnp.full((EP - E,), N, jnp.int32)
    rowp = jnp.concatenate([ei[0], pad])
    colp = jnp.concatenate([ei[1], pad])
    row_l1h = rowp.reshape(16, NCH1, CH)
    row_l1 = jnp.stack([row_l1h, row_l1h + NP])
    col_l1 = jnp.broadcast_to(colp.reshape(1, 16, NCH1, CH),
                              (2, 16, NCH1, CH))
    row_l2 = rowp.reshape(2, 16, NCH2, CH)
    col_l2 = colp.reshape(2, 16, NCH2, CH)

    zeros1 = jnp.zeros((NP,), jnp.float32)
    zeros2 = jnp.zeros((STRIPE, 128), jnp.float32)
    zeros2h = jnp.zeros((STRIPE, 64), jnp.float32)
    xp = jnp.pad(x, ((0, NP - N), (0, 0)))

    deg = _deg(col_l2, zeros1)
    ht0, ht1, dinv = _mm1_prescale(xp, W1, deg[0].reshape(NP, 1),
                                   deg[1].reshape(NP, 1))
    table1 = jnp.concatenate([ht0, ht1], axis=0)
    acc1 = _spmm1(row_l1, col_l1, table1, zeros2)
    t2a, t2b = _mid(acc1[0], acc1[1], ht0, ht1, dinv,
                    b1[:128].reshape(1, 128), b1[128:].reshape(1, 128), W2)
    table2 = jnp.concatenate([t2a, t2b], axis=0)
    del zeros2h
    out2 = _spmm2_fused(row_l1, col_l1, table2, dinv,
                        b2.reshape(2, 64))
    return jnp.concatenate([out2[0], out2[1]], axis=1)[:N]
